# baseline (speedup 1.0000x reference)
import functools

import jax
import jax.numpy as jnp
from jax import lax
from jax.experimental import pallas as pl
from jax.experimental.pallas import tpu as pltpu
from jax.experimental.pallas import tpu_sc as plsc

TV = 3072


@functools.cache
def _sc_gather(num_rows: int, vocab: int, embed: int):
    info = plsc.get_sparse_core_info()
    nw = info.num_cores * info.num_subcores
    bpw = num_rows // nw
    chunk = 128
    nch = (bpw + chunk - 1) // chunk
    mesh = plsc.VectorSubcoreMesh(core_axis_name="c", subcore_axis_name="s")

    @functools.partial(
        pl.kernel,
        mesh=mesh,
        compiler_params=pltpu.CompilerParams(use_tc_tiling_on_sc=False),
        out_type=jax.ShapeDtypeStruct((num_rows, embed), jnp.float32),
        scratch_types=[
            pltpu.VMEM((bpw,), jnp.int32),
            pltpu.VMEM((bpw, embed), jnp.float32),
            pltpu.SemaphoreType.DMA,
        ],
    )
    def gather_kernel(idx_hbm, table_hbm, out_hbm, idx_v, rows_v, sem):
        wid = lax.axis_index("s") * info.num_cores + lax.axis_index("c")
        base = wid * bpw
        pltpu.sync_copy(idx_hbm.at[pl.ds(base, bpw)], idx_v)
        copies = []
        for c in range(nch):
            off = c * chunk
            sz = min(chunk, bpw - off)
            copies.append(
                pltpu.async_copy(
                    table_hbm.at[idx_v.at[pl.ds(off, sz)]],
                    rows_v.at[pl.ds(off, sz)],
                    sem,
                )
            )
        for cp in copies:
            cp.wait()
        pltpu.sync_copy(rows_v, out_hbm.at[pl.ds(base, bpw)])

    return gather_kernel


def _mlp_softmax(gathered, W1, b1, W2, b2, batch, ctx2, embed):
    hidden = W1.shape[1]
    vocab = W2.shape[1]
    nj = (vocab + TV - 1) // TV
    nc2 = ctx2 // 2
    p2, p3 = nc2, nc2 + nj
    steps = nc2 + 2 * nj

    def vocab_j(c):
        return jnp.where(c < p3, jnp.maximum(c - p2, 0), c - p3)

    def body(x_ref, x2_ref, w1_ref, w12_ref, b1_ref, w2_ref, b2_ref, o_ref,
             acc_ref, h_ref, m_ref, s_ref, lse_ref):
        c = pl.program_id(0)

        @pl.when(c == 0)
        def _():
            acc_ref[...] = jnp.zeros((batch, hidden), jnp.float32)

        @pl.when(c < p2)
        def _():
            acc_ref[...] += jnp.dot(
                x_ref[...].astype(jnp.bfloat16),
                w1_ref[...].astype(jnp.bfloat16),
                preferred_element_type=jnp.float32,
            ) + jnp.dot(
                x2_ref[...].astype(jnp.bfloat16),
                w12_ref[...].astype(jnp.bfloat16),
                preferred_element_type=jnp.float32,
            )

        @pl.when(c == p2 - 1)
        def _():
            h_ref[...] = jnp.maximum(
                acc_ref[...] + b1_ref[...], 0.0
            ).astype(jnp.bfloat16)
            m_ref[...] = jnp.full((batch, 128), -jnp.inf, jnp.float32)
            s_ref[...] = jnp.zeros((batch, 128), jnp.float32)

        @pl.when((c >= p2) & (c < p3))
        def _():
            j = c - p2
            logits = (
                jnp.dot(
                    h_ref[...],
                    w2_ref[...].astype(jnp.bfloat16),
                    preferred_element_type=jnp.float32,
                )
                + b2_ref[...]
            )
            col = j * TV + lax.broadcasted_iota(jnp.int32, (1, TV), 1)
            logits = jnp.where(col < vocab, logits, -jnp.inf)
            chunks = [
                logits[:, k * 128 : (k + 1) * 128] for k in range(TV // 128)
            ]
            bm = chunks[0]
            for ch in chunks[1:]:
                bm = jnp.maximum(bm, ch)
            m_old = m_ref[...]
            m_new = jnp.maximum(m_old, bm)
            sval = s_ref[...] * jnp.exp(m_old - m_new)
            for ch in chunks:
                sval = sval + jnp.exp(ch - m_new)
            m_ref[...] = m_new
            s_ref[...] = sval

            @pl.when(c == p3 - 1)
            def _():
                big = jnp.max(m_new, axis=1, keepdims=True)
                tot = jnp.sum(
                    sval * jnp.exp(m_new - big), axis=1, keepdims=True
                )
                lse_ref[...] = big + jnp.log(tot)

        @pl.when(c >= p3)
        def _():
            logits = (
                jnp.dot(
                    h_ref[...],
                    w2_ref[...].astype(jnp.bfloat16),
                    preferred_element_type=jnp.float32,
                )
                + b2_ref[...]
            )
            o_ref[...] = logits - lse_ref[...]

    return pl.pallas_call(
        body,
        grid=(steps,),
        in_specs=[
            pl.BlockSpec((batch, embed), lambda c: (jnp.minimum(c, p2 - 1), 0)),
            pl.BlockSpec(
                (batch, embed), lambda c: (nc2 + jnp.minimum(c, p2 - 1), 0)
            ),
            pl.BlockSpec((embed, hidden), lambda c: (jnp.minimum(c, p2 - 1), 0)),
            pl.BlockSpec(
                (embed, hidden), lambda c: (nc2 + jnp.minimum(c, p2 - 1), 0)
            ),
            pl.BlockSpec((1, hidden), lambda c: (0, 0)),
            pl.BlockSpec((hidden, TV), lambda c: (0, vocab_j(c))),
            pl.BlockSpec((1, TV), lambda c: (0, vocab_j(c))),
        ],
        out_specs=pl.BlockSpec(
            (batch, TV), lambda c: (0, jnp.where(c < p3, 0, c - p3))
        ),
        out_shape=jax.ShapeDtypeStruct((batch, vocab), jnp.float32),
        scratch_shapes=[
            pltpu.VMEM((batch, hidden), jnp.float32),
            pltpu.VMEM((batch, hidden), jnp.bfloat16),
            pltpu.VMEM((batch, 128), jnp.float32),
            pltpu.VMEM((batch, 128), jnp.float32),
            pltpu.VMEM((batch, 1), jnp.float32),
        ],
    )(gathered, gathered, W1, W1, b1.reshape(1, hidden), W2,
      b2.reshape(1, vocab))


def kernel(inputs, emb, W1, b1, W2, b2):
    batch, ctx2 = inputs.shape
    vocab, embed = emb.shape
    idx = inputs.astype(jnp.int32).T.reshape(-1)
    gathered = _sc_gather(batch * ctx2, vocab, embed)(idx, emb)
    return _mlp_softmax(gathered, W1, b1, W2, b2, batch, ctx2, embed)
